# baseline (speedup 1.0000x reference)
import functools

import jax
import jax.numpy as jnp
from jax import lax
from jax.experimental import pallas as pl
from jax.experimental.pallas import tpu as pltpu
from jax.experimental.pallas import tpu_sc as plsc

B = 16384
D = 50
NUMERIC = 13
NC = 2
NS = 16
NW = NC * NS
BPW = B // NW
EPS = 1e-5
BM = 2048


def _sc_gather3(eidx, emb0, emb1, emb2):
    mesh = plsc.VectorSubcoreMesh(core_axis_name="c", subcore_axis_name="s")
    out_t = [jax.ShapeDtypeStruct((B * D,), jnp.float32)] * 3
    epw = BPW * D

    @functools.partial(
        pl.kernel,
        mesh=mesh,
        out_type=out_t,
        scratch_types=[
            pltpu.VMEM((epw,), jnp.int32),
            pltpu.VMEM((epw,), jnp.float32),
            pltpu.SemaphoreType.DMA,
        ],
    )
    def k(i0_hbm, i1_hbm, i2_hbm, t0, t1, t2, o0, o1, o2, idx_v, rows_v, sem):
        wid = lax.axis_index("s") * NC + lax.axis_index("c")
        base = wid * epw
        for ix, tbl, out in ((i0_hbm, t0, o0), (i1_hbm, t1, o1),
                             (i2_hbm, t2, o2)):
            pltpu.sync_copy(ix.at[pl.ds(base, epw)], idx_v)
            pltpu.async_copy(tbl.at[idx_v], rows_v, sem).wait()
            pltpu.sync_copy(rows_v, out.at[pl.ds(base, epw)])

    return k(eidx[0], eidx[1], eidx[2], emb0.reshape(-1), emb1.reshape(-1),
             emb2.reshape(-1))


def _mlp_body(e0_ref, e1_ref, e2_ref, xn_ref, gn_ref, bn_ref, w1a_ref,
              w1b_ref, w1c_ref, w1d_ref, b1_ref, g1_ref, be1_ref, w2_ref,
              b2_ref, g2_ref, be2_ref, w3_ref, b3_ref, o_ref):
    inv = (1.0 + EPS) ** -0.5
    f32 = jnp.float32
    xnb = xn_ref[...] * (gn_ref[...] * inv) + bn_ref[...]
    h = jnp.dot(e0_ref[...], w1a_ref[...], preferred_element_type=f32)
    h = h + jnp.dot(e1_ref[...], w1b_ref[...], preferred_element_type=f32)
    h = h + jnp.dot(e2_ref[...], w1c_ref[...], preferred_element_type=f32)
    h = h + jnp.dot(xnb, w1d_ref[...], preferred_element_type=f32)
    h = jnp.maximum(h + b1_ref[...], 0.0) * (g1_ref[...] * inv) + be1_ref[...]
    h = jnp.dot(h, w2_ref[...], preferred_element_type=f32)
    h = jnp.maximum(h + b2_ref[...], 0.0) * (g2_ref[...] * inv) + be2_ref[...]
    o_ref[...] = jnp.dot(h, w3_ref[...], preferred_element_type=f32) + b3_ref[...]


def _tc_mlp(e0, e1, e2, xn, g_num, b_num, W1a, W1b, W1c, W1d, b1, g1, be1,
            W2, b2, g2, be2, W3, b3):
    grid = (B // BM,)
    full = lambda shape: pl.BlockSpec(shape, lambda i: tuple(0 for _ in shape))
    return pl.pallas_call(
        _mlp_body,
        grid=grid,
        in_specs=[
            pl.BlockSpec((BM, D), lambda i: (i, 0)),
            pl.BlockSpec((BM, D), lambda i: (i, 0)),
            pl.BlockSpec((BM, D), lambda i: (i, 0)),
            pl.BlockSpec((BM, NUMERIC), lambda i: (i, 0)),
            full(g_num.shape), full(b_num.shape),
            full(W1a.shape), full(W1b.shape), full(W1c.shape), full(W1d.shape),
            full(b1.shape), full(g1.shape), full(be1.shape),
            full(W2.shape), full(b2.shape), full(g2.shape), full(be2.shape),
            full(W3.shape), full(b3.shape),
        ],
        out_specs=pl.BlockSpec((BM, 1), lambda i: (i, 0)),
        out_shape=jax.ShapeDtypeStruct((B, 1), jnp.float32),
    )(e0, e1, e2, xn, g_num, b_num, W1a, W1b, W1c, W1d, b1, g1, be1, W2, b2,
      g2, be2, W3, b3)


def kernel(x, emb0, emb1, emb2, g_num, b_num, W1, b1, g1, be1, W2, b2, g2,
           be2, W3, b3):
    idx = x[:, :3].astype(jnp.int32)
    xn = x[:, 3:]
    eidx = (idx.T[:, :, None] * D + jnp.arange(D, dtype=jnp.int32)).reshape(3, B * D)
    e0, e1, e2 = _sc_gather3(eidx, emb0, emb1, emb2)
    e0, e1, e2 = (e.reshape(B, D) for e in (e0, e1, e2))
    W1a, W1b, W1c, W1d = W1[:D], W1[D:2 * D], W1[2 * D:3 * D], W1[3 * D:]
    return _tc_mlp(e0, e1, e2, xn, g_num, b_num, W1a, W1b, W1c, W1d, b1, g1,
                   be1, W2, b2, g2, be2, W3, b3)


# speedup vs baseline: 1.3528x; 1.3528x over previous
import functools

import jax
import jax.numpy as jnp
from jax import lax
from jax.experimental import pallas as pl
from jax.experimental.pallas import tpu as pltpu
from jax.experimental.pallas import tpu_sc as plsc

B = 16384
D = 50
NUMERIC = 13
NC = 2
NS = 16
NW = NC * NS
BPW = B // NW
EPS = 1e-5
BM = 2048


def _sc_gather3(idx0, idx1, idx2, emb0, emb1, emb2):
    mesh = plsc.VectorSubcoreMesh(core_axis_name="c", subcore_axis_name="s")
    out_t = [jax.ShapeDtypeStruct((B, D), jnp.float32)] * 3

    @functools.partial(
        pl.kernel,
        mesh=mesh,
        out_type=out_t,
        scratch_types=[
            pltpu.VMEM((BPW,), jnp.int32),
            pltpu.SemaphoreType.DMA,
        ],
    )
    def k(i0_hbm, i1_hbm, i2_hbm, t0, t1, t2, o0, o1, o2, idx_v, sem):
        wid = lax.axis_index("s") * NC + lax.axis_index("c")
        base = wid * BPW
        for ix, tbl, out in ((i0_hbm, t0, o0), (i1_hbm, t1, o1),
                             (i2_hbm, t2, o2)):
            pltpu.sync_copy(ix.at[pl.ds(base, BPW)], idx_v)

            @pl.loop(0, BPW, unroll=8)
            def _(i):
                r = idx_v[pl.ds(i, 1)][0]
                pltpu.async_copy(tbl.at[pl.ds(r, 1), :],
                                 out.at[pl.ds(base + i, 1), :], sem)

            pltpu.make_async_copy(out.at[pl.ds(base, BPW), :],
                                  out.at[pl.ds(base, BPW), :], sem).wait()

    return k(idx0, idx1, idx2, emb0, emb1, emb2)


def _mlp_body(e0_ref, e1_ref, e2_ref, xn_ref, gn_ref, bn_ref, w1a_ref,
              w1b_ref, w1c_ref, w1d_ref, b1_ref, g1_ref, be1_ref, w2_ref,
              b2_ref, g2_ref, be2_ref, w3_ref, b3_ref, o_ref):
    inv = (1.0 + EPS) ** -0.5
    f32 = jnp.float32
    xnb = xn_ref[...] * (gn_ref[...] * inv) + bn_ref[...]
    h = jnp.dot(e0_ref[...], w1a_ref[...], preferred_element_type=f32)
    h = h + jnp.dot(e1_ref[...], w1b_ref[...], preferred_element_type=f32)
    h = h + jnp.dot(e2_ref[...], w1c_ref[...], preferred_element_type=f32)
    h = h + jnp.dot(xnb, w1d_ref[...], preferred_element_type=f32)
    h = jnp.maximum(h + b1_ref[...], 0.0) * (g1_ref[...] * inv) + be1_ref[...]
    h = jnp.dot(h, w2_ref[...], preferred_element_type=f32)
    h = jnp.maximum(h + b2_ref[...], 0.0) * (g2_ref[...] * inv) + be2_ref[...]
    o_ref[...] = jnp.dot(h, w3_ref[...], preferred_element_type=f32) + b3_ref[...]


def _tc_mlp(e0, e1, e2, xn, g_num, b_num, W1a, W1b, W1c, W1d, b1, g1, be1,
            W2, b2, g2, be2, W3, b3):
    grid = (B // BM,)
    full = lambda shape: pl.BlockSpec(shape, lambda i: tuple(0 for _ in shape))
    return pl.pallas_call(
        _mlp_body,
        grid=grid,
        in_specs=[
            pl.BlockSpec((BM, D), lambda i: (i, 0)),
            pl.BlockSpec((BM, D), lambda i: (i, 0)),
            pl.BlockSpec((BM, D), lambda i: (i, 0)),
            pl.BlockSpec((BM, NUMERIC), lambda i: (i, 0)),
            full(g_num.shape), full(b_num.shape),
            full(W1a.shape), full(W1b.shape), full(W1c.shape), full(W1d.shape),
            full(b1.shape), full(g1.shape), full(be1.shape),
            full(W2.shape), full(b2.shape), full(g2.shape), full(be2.shape),
            full(W3.shape), full(b3.shape),
        ],
        out_specs=pl.BlockSpec((BM, 1), lambda i: (i, 0)),
        out_shape=jax.ShapeDtypeStruct((B, 1), jnp.float32),
    )(e0, e1, e2, xn, g_num, b_num, W1a, W1b, W1c, W1d, b1, g1, be1, W2, b2,
      g2, be2, W3, b3)


def kernel(x, emb0, emb1, emb2, g_num, b_num, W1, b1, g1, be1, W2, b2, g2,
           be2, W3, b3):
    idx = x[:, :3].astype(jnp.int32)
    xn = x[:, 3:]
    e0, e1, e2 = _sc_gather3(idx[:, 0], idx[:, 1], idx[:, 2], emb0, emb1,
                             emb2)
    W1a, W1b, W1c, W1d = W1[:D], W1[D:2 * D], W1[2 * D:3 * D], W1[3 * D:]
    return _tc_mlp(e0, e1, e2, xn, g_num, b_num, W1a, W1b, W1c, W1d, b1, g1,
                   be1, W2, b2, g2, be2, W3, b3)


# speedup vs baseline: 3.3753x; 2.4951x over previous
import functools

import jax
import jax.numpy as jnp
from jax import lax
from jax.experimental import pallas as pl
from jax.experimental.pallas import tpu as pltpu
from jax.experimental.pallas import tpu_sc as plsc

B = 16384
D = 50
NUMERIC = 13
NC = 2
NS = 16
NW = NC * NS
BPW = B // NW
EPS = 1e-5
BM = 2048


NQ = 4


def _sc_gather3(idx0, idx1, idx2, emb0, emb1, emb2):
    mesh = plsc.VectorSubcoreMesh(core_axis_name="c", subcore_axis_name="s")
    out_t = [jax.ShapeDtypeStruct((B, D), jnp.float32)] * 3

    @functools.partial(
        pl.kernel,
        mesh=mesh,
        out_type=out_t,
        scratch_types=[
            pltpu.VMEM((BPW,), jnp.int32),
            pltpu.VMEM((BPW, D), jnp.float32),
            pltpu.SemaphoreType.DMA,
            pltpu.SemaphoreType.DMA,
            pltpu.SemaphoreType.DMA,
            pltpu.SemaphoreType.DMA,
        ],
    )
    def k(i0_hbm, i1_hbm, i2_hbm, t0, t1, t2, o0, o1, o2, idx_v, rows_v,
          *sems):
        wid = lax.axis_index("s") * NC + lax.axis_index("c")
        base = wid * BPW
        for ix, tbl, out in ((i0_hbm, t0, o0), (i1_hbm, t1, o1),
                             (i2_hbm, t2, o2)):
            pltpu.sync_copy(ix.at[pl.ds(base, BPW)], idx_v)

            @pl.loop(0, BPW, step=NQ)
            def _(i):
                for q in range(NQ):
                    r = idx_v[pl.ds(i + q, 1)][0]
                    pltpu.async_copy(tbl.at[pl.ds(r, 1), :],
                                     rows_v.at[pl.ds(i + q, 1), :], sems[q])

            for q in range(NQ):
                pltpu.make_async_copy(
                    out.at[pl.ds(base, BPW // NQ), :],
                    rows_v.at[pl.ds(0, BPW // NQ), :], sems[q]).wait()
            pltpu.sync_copy(rows_v, out.at[pl.ds(base, BPW), :])

    return k(idx0, idx1, idx2, emb0, emb1, emb2)


def _mlp_body(e0_ref, e1_ref, e2_ref, xn_ref, gn_ref, bn_ref, w1a_ref,
              w1b_ref, w1c_ref, w1d_ref, b1_ref, g1_ref, be1_ref, w2_ref,
              b2_ref, g2_ref, be2_ref, w3_ref, b3_ref, o_ref):
    inv = (1.0 + EPS) ** -0.5
    f32 = jnp.float32
    xnb = xn_ref[...] * (gn_ref[...] * inv) + bn_ref[...]
    h = jnp.dot(e0_ref[...], w1a_ref[...], preferred_element_type=f32)
    h = h + jnp.dot(e1_ref[...], w1b_ref[...], preferred_element_type=f32)
    h = h + jnp.dot(e2_ref[...], w1c_ref[...], preferred_element_type=f32)
    h = h + jnp.dot(xnb, w1d_ref[...], preferred_element_type=f32)
    h = jnp.maximum(h + b1_ref[...], 0.0) * (g1_ref[...] * inv) + be1_ref[...]
    h = jnp.dot(h, w2_ref[...], preferred_element_type=f32)
    h = jnp.maximum(h + b2_ref[...], 0.0) * (g2_ref[...] * inv) + be2_ref[...]
    o_ref[...] = jnp.dot(h, w3_ref[...], preferred_element_type=f32) + b3_ref[...]


def _tc_mlp(e0, e1, e2, xn, g_num, b_num, W1a, W1b, W1c, W1d, b1, g1, be1,
            W2, b2, g2, be2, W3, b3):
    grid = (B // BM,)
    full = lambda shape: pl.BlockSpec(shape, lambda i: tuple(0 for _ in shape))
    return pl.pallas_call(
        _mlp_body,
        grid=grid,
        in_specs=[
            pl.BlockSpec((BM, D), lambda i: (i, 0)),
            pl.BlockSpec((BM, D), lambda i: (i, 0)),
            pl.BlockSpec((BM, D), lambda i: (i, 0)),
            pl.BlockSpec((BM, NUMERIC), lambda i: (i, 0)),
            full(g_num.shape), full(b_num.shape),
            full(W1a.shape), full(W1b.shape), full(W1c.shape), full(W1d.shape),
            full(b1.shape), full(g1.shape), full(be1.shape),
            full(W2.shape), full(b2.shape), full(g2.shape), full(be2.shape),
            full(W3.shape), full(b3.shape),
        ],
        out_specs=pl.BlockSpec((BM, 1), lambda i: (i, 0)),
        out_shape=jax.ShapeDtypeStruct((B, 1), jnp.float32),
    )(e0, e1, e2, xn, g_num, b_num, W1a, W1b, W1c, W1d, b1, g1, be1, W2, b2,
      g2, be2, W3, b3)


def kernel(x, emb0, emb1, emb2, g_num, b_num, W1, b1, g1, be1, W2, b2, g2,
           be2, W3, b3):
    idx = x[:, :3].astype(jnp.int32)
    xn = x[:, 3:]
    e0, e1, e2 = _sc_gather3(idx[:, 0], idx[:, 1], idx[:, 2], emb0, emb1,
                             emb2)
    W1a, W1b, W1c, W1d = W1[:D], W1[D:2 * D], W1[2 * D:3 * D], W1[3 * D:]
    return _tc_mlp(e0, e1, e2, xn, g_num, b_num, W1a, W1b, W1c, W1d, b1, g1,
                   be1, W2, b2, g2, be2, W3, b3)


# speedup vs baseline: 3.3837x; 1.0025x over previous
import functools

import jax
import jax.numpy as jnp
from jax import lax
from jax.experimental import pallas as pl
from jax.experimental.pallas import tpu as pltpu
from jax.experimental.pallas import tpu_sc as plsc

B = 16384
D = 50
NUMERIC = 13
NC = 2
NS = 16
NW = NC * NS
BPW = B // NW
EPS = 1e-5
BM = 2048


NQ = 8


def _sc_gather3(idx0, idx1, idx2, emb0, emb1, emb2):
    mesh = plsc.VectorSubcoreMesh(core_axis_name="c", subcore_axis_name="s")
    out_t = [jax.ShapeDtypeStruct((B, D), jnp.float32)] * 3

    @functools.partial(
        pl.kernel,
        mesh=mesh,
        out_type=out_t,
        scratch_types=[
            pltpu.VMEM((BPW,), jnp.int32),
            pltpu.VMEM((BPW, D), jnp.float32),
        ] + [pltpu.SemaphoreType.DMA] * NQ,
    )
    def k(i0_hbm, i1_hbm, i2_hbm, t0, t1, t2, o0, o1, o2, idx_v, rows_v,
          *sems):
        wid = lax.axis_index("s") * NC + lax.axis_index("c")
        base = wid * BPW
        for ix, tbl, out in ((i0_hbm, t0, o0), (i1_hbm, t1, o1),
                             (i2_hbm, t2, o2)):
            pltpu.sync_copy(ix.at[pl.ds(base, BPW)], idx_v)

            @pl.loop(0, BPW, step=NQ)
            def _(i):
                for q in range(NQ):
                    r = idx_v[pl.ds(i + q, 1)][0]
                    pltpu.async_copy(tbl.at[pl.ds(r, 1), :],
                                     rows_v.at[pl.ds(i + q, 1), :], sems[q])

            for q in range(NQ):
                pltpu.make_async_copy(
                    out.at[pl.ds(base, BPW // NQ), :],
                    rows_v.at[pl.ds(0, BPW // NQ), :], sems[q]).wait()
            pltpu.sync_copy(rows_v, out.at[pl.ds(base, BPW), :])

    return k(idx0, idx1, idx2, emb0, emb1, emb2)


def _mlp_body(e0_ref, e1_ref, e2_ref, xn_ref, gn_ref, bn_ref, w1a_ref,
              w1b_ref, w1c_ref, w1d_ref, b1_ref, g1_ref, be1_ref, w2_ref,
              b2_ref, g2_ref, be2_ref, w3_ref, b3_ref, o_ref):
    inv = (1.0 + EPS) ** -0.5
    f32 = jnp.float32
    xnb = xn_ref[...] * (gn_ref[...] * inv) + bn_ref[...]
    h = jnp.dot(e0_ref[...], w1a_ref[...], preferred_element_type=f32)
    h = h + jnp.dot(e1_ref[...], w1b_ref[...], preferred_element_type=f32)
    h = h + jnp.dot(e2_ref[...], w1c_ref[...], preferred_element_type=f32)
    h = h + jnp.dot(xnb, w1d_ref[...], preferred_element_type=f32)
    h = jnp.maximum(h + b1_ref[...], 0.0) * (g1_ref[...] * inv) + be1_ref[...]
    h = jnp.dot(h, w2_ref[...], preferred_element_type=f32)
    h = jnp.maximum(h + b2_ref[...], 0.0) * (g2_ref[...] * inv) + be2_ref[...]
    o_ref[...] = jnp.dot(h, w3_ref[...], preferred_element_type=f32) + b3_ref[...]


def _tc_mlp(e0, e1, e2, xn, g_num, b_num, W1a, W1b, W1c, W1d, b1, g1, be1,
            W2, b2, g2, be2, W3, b3):
    grid = (B // BM,)
    full = lambda shape: pl.BlockSpec(shape, lambda i: tuple(0 for _ in shape))
    return pl.pallas_call(
        _mlp_body,
        grid=grid,
        in_specs=[
            pl.BlockSpec((BM, D), lambda i: (i, 0)),
            pl.BlockSpec((BM, D), lambda i: (i, 0)),
            pl.BlockSpec((BM, D), lambda i: (i, 0)),
            pl.BlockSpec((BM, NUMERIC), lambda i: (i, 0)),
            full(g_num.shape), full(b_num.shape),
            full(W1a.shape), full(W1b.shape), full(W1c.shape), full(W1d.shape),
            full(b1.shape), full(g1.shape), full(be1.shape),
            full(W2.shape), full(b2.shape), full(g2.shape), full(be2.shape),
            full(W3.shape), full(b3.shape),
        ],
        out_specs=pl.BlockSpec((BM, 1), lambda i: (i, 0)),
        out_shape=jax.ShapeDtypeStruct((B, 1), jnp.float32),
    )(e0, e1, e2, xn, g_num, b_num, W1a, W1b, W1c, W1d, b1, g1, be1, W2, b2,
      g2, be2, W3, b3)


def kernel(x, emb0, emb1, emb2, g_num, b_num, W1, b1, g1, be1, W2, b2, g2,
           be2, W3, b3):
    idx = x[:, :3].astype(jnp.int32)
    xn = x[:, 3:]
    e0, e1, e2 = _sc_gather3(idx[:, 0], idx[:, 1], idx[:, 2], emb0, emb1,
                             emb2)
    W1a, W1b, W1c, W1d = W1[:D], W1[D:2 * D], W1[2 * D:3 * D], W1[3 * D:]
    return _tc_mlp(e0, e1, e2, xn, g_num, b_num, W1a, W1b, W1c, W1d, b1, g1,
                   be1, W2, b2, g2, be2, W3, b3)


# speedup vs baseline: 3.4278x; 1.0130x over previous
import functools

import jax
import jax.numpy as jnp
from jax import lax
from jax.experimental import pallas as pl
from jax.experimental.pallas import tpu as pltpu
from jax.experimental.pallas import tpu_sc as plsc

B = 16384
D = 50
NUMERIC = 13
NC = 2
NS = 16
NW = NC * NS
BPW = B // NW
EPS = 1e-5
BM = 2048
NQ = 8


def _sc_gather(idxs, tbls):
    n = len(tbls)
    mesh = plsc.VectorSubcoreMesh(core_axis_name="c", subcore_axis_name="s")
    out_t = [jax.ShapeDtypeStruct((B, D), jnp.float32)] * n

    @functools.partial(
        pl.kernel,
        mesh=mesh,
        out_type=out_t,
        scratch_types=[
            pltpu.VMEM((BPW,), jnp.int32),
            pltpu.VMEM((BPW, D), jnp.float32),
        ] + [pltpu.SemaphoreType.DMA] * NQ,
    )
    def k(*args):
        ixs, rest = args[:n], args[n:]
        ts, rest = rest[:n], rest[n:]
        outs, rest = rest[:n], rest[n:]
        idx_v, rows_v, *sems = rest
        wid = lax.axis_index("s") * NC + lax.axis_index("c")
        base = wid * BPW
        for ix, tbl, out in zip(ixs, ts, outs):
            pltpu.sync_copy(ix.at[pl.ds(base, BPW)], idx_v)

            @pl.loop(0, BPW, step=NQ)
            def _(i):
                for q in range(NQ):
                    r = idx_v[pl.ds(i + q, 1)][0]
                    pltpu.async_copy(tbl.at[pl.ds(r, 1), :],
                                     rows_v.at[pl.ds(i + q, 1), :], sems[q])

            for q in range(NQ):
                pltpu.make_async_copy(
                    out.at[pl.ds(base, BPW // NQ), :],
                    rows_v.at[pl.ds(0, BPW // NQ), :], sems[q]).wait()
            pltpu.sync_copy(rows_v, out.at[pl.ds(base, BPW), :])

    return k(*idxs, *tbls)


def _mlp_body(e0_ref, e1_ref, e2_ref, xn_ref, gn_ref, bn_ref, w1a_ref,
              w1b_ref, w1c_ref, w1d_ref, b1_ref, g1_ref, be1_ref, w2_ref,
              b2_ref, g2_ref, be2_ref, w3_ref, b3_ref, o_ref):
    inv = (1.0 + EPS) ** -0.5
    f32 = jnp.float32
    xnb = xn_ref[...] * (gn_ref[...] * inv) + bn_ref[...]
    h = jnp.dot(e0_ref[...], w1a_ref[...], preferred_element_type=f32)
    h = h + jnp.dot(e1_ref[...], w1b_ref[...], preferred_element_type=f32)
    h = h + jnp.dot(e2_ref[...], w1c_ref[...], preferred_element_type=f32)
    h = h + jnp.dot(xnb, w1d_ref[...], preferred_element_type=f32)
    h = jnp.maximum(h + b1_ref[...], 0.0) * (g1_ref[...] * inv) + be1_ref[...]
    h = jnp.dot(h, w2_ref[...], preferred_element_type=f32)
    h = jnp.maximum(h + b2_ref[...], 0.0) * (g2_ref[...] * inv) + be2_ref[...]
    o_ref[...] = jnp.dot(h, w3_ref[...], preferred_element_type=f32) + b3_ref[...]


def _tc_mlp(e0, e1, e2, xn, g_num, b_num, W1a, W1b, W1c, W1d, b1, g1, be1,
            W2, b2, g2, be2, W3, b3):
    grid = (B // BM,)
    full = lambda shape: pl.BlockSpec(shape, lambda i: tuple(0 for _ in shape))
    return pl.pallas_call(
        _mlp_body,
        grid=grid,
        in_specs=[
            pl.BlockSpec((BM, D), lambda i: (i, 0)),
            pl.BlockSpec((BM, D), lambda i: (i, 0)),
            pl.BlockSpec((BM, D), lambda i: (i, 0)),
            pl.BlockSpec((BM, NUMERIC), lambda i: (i, 0)),
            full(g_num.shape), full(b_num.shape),
            full(W1a.shape), full(W1b.shape), full(W1c.shape), full(W1d.shape),
            full(b1.shape), full(g1.shape), full(be1.shape),
            full(W2.shape), full(b2.shape), full(g2.shape), full(be2.shape),
            full(W3.shape), full(b3.shape),
        ],
        out_specs=pl.BlockSpec((BM, 1), lambda i: (i, 0)),
        out_shape=jax.ShapeDtypeStruct((B, 1), jnp.float32),
    )(e0, e1, e2, xn, g_num, b_num, W1a, W1b, W1c, W1d, b1, g1, be1, W2, b2,
      g2, be2, W3, b3)


def kernel(x, emb0, emb1, emb2, g_num, b_num, W1, b1, g1, be1, W2, b2, g2,
           be2, W3, b3):
    idx = x[:, :3].astype(jnp.int32)
    xn = x[:, 3:]
    (e1, e2) = _sc_gather([idx[:, 1], idx[:, 2]], [emb1, emb2])
    (e0,) = _sc_gather([idx[:, 0]], [emb0])
    W1a, W1b, W1c, W1d = W1[:D], W1[D:2 * D], W1[2 * D:3 * D], W1[3 * D:]
    return _tc_mlp(e0, e1, e2, xn, g_num, b_num, W1a, W1b, W1c, W1d, b1, g1,
                   be1, W2, b2, g2, be2, W3, b3)
